# baseline (speedup 1.0000x reference)
import functools

import jax
import jax.numpy as jnp
from jax import lax
from jax.experimental import pallas as pl
from jax.experimental.pallas import tpu as pltpu
from jax.experimental.pallas import tpu_sc as plsc

N_NODES = 10000
N_EDGES = 320000
D_FEAT = 128
NS = 16
NC = 2
NW = NS * NC
K = 64
NCHUNK = 160
E_PAD = NW * NCHUNK * K
EPT = NCHUNK * K
NBUF = 4
SUPER = 40
NSUPER = NCHUNK // SUPER
NGRP = SUPER // NBUF
NW_OUT = 10
ROWS_PT = N_NODES // NW_OUT


def _sc_body(ego, rowi, coli, vals, out,
             colb, rowb, valb, gbuf, acc, gsem, ssem):
    core = lax.axis_index("c")
    tid = lax.axis_index("s")
    wid = core * NS + tid

    def zrow(r, carry):
        for j in range(D_FEAT // 16):
            gbuf[r, pl.ds(j * 16, 16)] = jnp.zeros((16,), jnp.float32)
        return carry

    lax.fori_loop(0, NBUF * K, zrow, 0)

    @pl.when(tid < NW_OUT)
    def _zero():
        for off, nr in ((0, 256), (256, 256), (512, 256), (768, 232)):
            pltpu.sync_copy(gbuf.at[pl.ds(0, nr)],
                            acc.at[pl.ds(tid * ROWS_PT + off, nr)])

    plsc.subcore_barrier()

    def superstep(s, carry):
        pltpu.sync_copy(coli.at[wid, pl.ds(s * SUPER, SUPER)], colb)
        pltpu.sync_copy(rowi.at[wid, pl.ds(s * SUPER, SUPER)], rowb)
        pltpu.sync_copy(vals.at[wid, pl.ds(s * SUPER, SUPER)], valb)

        def group(g, c1):
            gathers = []
            for j in range(NBUF):
                gathers.append(pltpu.async_copy(
                    ego.at[colb.at[g * NBUF + j]],
                    gbuf.at[pl.ds(j * K, K)], gsem.at[j]))
            scatters = []
            for j in range(NBUF):
                gathers[j].wait()

                def scale(gg, c2, j=j):
                    v16 = valb[g * NBUF + j, pl.ds(gg * 16, 16)]
                    for i2 in range(16):
                        v = v16[i2]
                        e = j * K + gg * 16 + i2
                        for q in range(D_FEAT // 16):
                            gbuf[e, pl.ds(q * 16, 16)] = (
                                gbuf[e, pl.ds(q * 16, 16)] * v)
                    return c2

                lax.fori_loop(0, K // 16, scale, 0)
                scatters.append(pltpu.async_copy(
                    gbuf.at[pl.ds(j * K, K)],
                    acc.at[rowb.at[g * NBUF + j]], ssem.at[j], add=True))
            for j in range(NBUF):
                scatters[j].wait()
            return c1

        lax.fori_loop(0, NGRP, group, 0)
        return carry

    lax.fori_loop(0, NSUPER, superstep, 0)

    plsc.subcore_barrier()

    @pl.when(tid < NW_OUT)
    def _writeout():
        pltpu.sync_copy(acc.at[pl.ds(tid * ROWS_PT, ROWS_PT)],
                        out.at[core, pl.ds(tid * ROWS_PT, ROWS_PT)])


def _combine_body(p_ref, o_ref):
    o_ref[...] = p_ref[0] + p_ref[1]


@jax.jit
def kernel(ego_embeddings, edge_index, adj_values):
    pad = E_PAD - N_EDGES
    rowi = jnp.concatenate(
        [edge_index[0], jnp.zeros((pad,), jnp.int32)]).reshape(NW, NCHUNK, K)
    coli = jnp.concatenate(
        [edge_index[1], jnp.zeros((pad,), jnp.int32)]).reshape(NW, NCHUNK, K)
    vals = jnp.concatenate(
        [adj_values, jnp.zeros((pad,), jnp.float32)]).reshape(NW, NCHUNK, K)

    mesh = plsc.VectorSubcoreMesh(core_axis_name="c", subcore_axis_name="s")
    partials = pl.kernel(
        _sc_body,
        out_type=jax.ShapeDtypeStruct((NC, N_NODES, D_FEAT), jnp.float32),
        mesh=mesh,
        scratch_types=[
            pltpu.VMEM((SUPER, K), jnp.int32),
            pltpu.VMEM((SUPER, K), jnp.int32),
            pltpu.VMEM((SUPER, K), jnp.float32),
            pltpu.VMEM((NBUF * K, D_FEAT), jnp.float32),
            pltpu.VMEM_SHARED((N_NODES, D_FEAT), jnp.float32),
            pltpu.SemaphoreType.DMA((NBUF,)),
            pltpu.SemaphoreType.DMA((NBUF,)),
        ],
    )(ego_embeddings, rowi, coli, vals)

    rows_blk = 2000
    return pl.pallas_call(
        _combine_body,
        grid=(N_NODES // rows_blk,),
        in_specs=[pl.BlockSpec((NC, rows_blk, D_FEAT), lambda i: (0, i, 0))],
        out_specs=pl.BlockSpec((rows_blk, D_FEAT), lambda i: (i, 0)),
        out_shape=jax.ShapeDtypeStruct((N_NODES, D_FEAT), jnp.float32),
    )(partials)


# speedup vs baseline: 2.7599x; 2.7599x over previous
import functools

import jax
import jax.numpy as jnp
from jax import lax
from jax.experimental import pallas as pl
from jax.experimental.pallas import tpu as pltpu
from jax.experimental.pallas import tpu_sc as plsc

N_NODES = 10000
N_EDGES = 320000
D_FEAT = 128
NS = 16
NC = 2
NW = NS * NC
K = 64
NCHUNK = 160
E_PAD = NW * NCHUNK * K
EPT = NCHUNK * K
NBUF = 4
SUPER = 40
NSUPER = NCHUNK // SUPER
NGRP = SUPER // NBUF
NW_OUT = 10
ROWS_PT = N_NODES // NW_OUT


def _sc_body(ego, rowi, coli, vals, out,
             colb, rowb, valb, gbuf, acc, gsem, ssem):
    core = lax.axis_index("c")
    tid = lax.axis_index("s")
    wid = core * NS + tid

    def zrow(r, carry):
        for j in range(D_FEAT // 16):
            gbuf[r, pl.ds(j * 16, 16)] = jnp.zeros((16,), jnp.float32)
        return carry

    lax.fori_loop(0, NBUF * K, zrow, 0)

    @pl.when(tid < NW_OUT)
    def _zero():
        for off, nr in ((0, 256), (256, 256), (512, 256), (768, 232)):
            pltpu.sync_copy(gbuf.at[pl.ds(0, nr)],
                            acc.at[pl.ds(tid * ROWS_PT + off, nr)])

    plsc.subcore_barrier()

    def superstep(s, carry):
        pltpu.sync_copy(coli.at[wid, pl.ds(s * SUPER, SUPER)], colb)
        pltpu.sync_copy(rowi.at[wid, pl.ds(s * SUPER, SUPER)], rowb)
        pltpu.sync_copy(vals.at[wid, pl.ds(s * SUPER, SUPER)], valb)

        def group(g, c1):
            gathers = []
            for j in range(NBUF):
                gathers.append(pltpu.async_copy(
                    ego.at[colb.at[g * NBUF + j]],
                    gbuf.at[pl.ds(j * K, K)], gsem.at[j]))
            scatters = []
            for j in range(NBUF):
                gathers[j].wait()

                def scale(gg, c2, j=j):
                    v16 = valb[g * NBUF + j, pl.ds(gg * 16, 16)]
                    for i2 in range(16):
                        v = v16[i2]
                        e = j * K + gg * 16 + i2
                        for q in range(D_FEAT // 16):
                            gbuf[e, pl.ds(q * 16, 16)] = (
                                gbuf[e, pl.ds(q * 16, 16)] * v)
                    return c2

                lax.fori_loop(0, K // 16, scale, 0)
                scatters.append(pltpu.async_copy(
                    gbuf.at[pl.ds(j * K, K)],
                    acc.at[rowb.at[g * NBUF + j]], ssem.at[j], add=True))
            for j in range(NBUF):
                scatters[j].wait()
            return c1

        lax.fori_loop(0, NGRP, group, 0)
        return carry

    lax.fori_loop(0, NSUPER, superstep, 0)

    plsc.subcore_barrier()

    @pl.when(tid < NW_OUT)
    def _writeout():
        pltpu.sync_copy(acc.at[pl.ds(tid * ROWS_PT, ROWS_PT)],
                        out.at[core, pl.ds(tid * ROWS_PT, ROWS_PT)])


def _combine_body(p_ref, o_ref):
    o_ref[...] = p_ref[0] + p_ref[1]


@jax.jit
def kernel(ego_embeddings, edge_index, adj_values):
    pad = E_PAD - N_EDGES
    spread = (jnp.arange(pad, dtype=jnp.int32) * 8) % N_NODES
    rowi = jnp.concatenate(
        [edge_index[0], spread]).reshape(NW, NCHUNK, K)
    coli = jnp.concatenate(
        [edge_index[1], spread]).reshape(NW, NCHUNK, K)
    vals = jnp.concatenate(
        [adj_values, jnp.zeros((pad,), jnp.float32)]).reshape(NW, NCHUNK, K)

    mesh = plsc.VectorSubcoreMesh(core_axis_name="c", subcore_axis_name="s")
    partials = pl.kernel(
        _sc_body,
        out_type=jax.ShapeDtypeStruct((NC, N_NODES, D_FEAT), jnp.float32),
        mesh=mesh,
        scratch_types=[
            pltpu.VMEM((SUPER, K), jnp.int32),
            pltpu.VMEM((SUPER, K), jnp.int32),
            pltpu.VMEM((SUPER, K), jnp.float32),
            pltpu.VMEM((NBUF * K, D_FEAT), jnp.float32),
            pltpu.VMEM_SHARED((N_NODES, D_FEAT), jnp.float32),
            pltpu.SemaphoreType.DMA((NBUF,)),
            pltpu.SemaphoreType.DMA((NBUF,)),
        ],
    )(ego_embeddings, rowi, coli, vals)

    rows_blk = 2000
    return pl.pallas_call(
        _combine_body,
        grid=(N_NODES // rows_blk,),
        in_specs=[pl.BlockSpec((NC, rows_blk, D_FEAT), lambda i: (0, i, 0))],
        out_specs=pl.BlockSpec((rows_blk, D_FEAT), lambda i: (i, 0)),
        out_shape=jax.ShapeDtypeStruct((N_NODES, D_FEAT), jnp.float32),
    )(partials)


# speedup vs baseline: 2.7736x; 1.0050x over previous
import functools

import jax
import jax.numpy as jnp
from jax import lax
from jax.experimental import pallas as pl
from jax.experimental.pallas import tpu as pltpu
from jax.experimental.pallas import tpu_sc as plsc

N_NODES = 10000
N_EDGES = 320000
D_FEAT = 128
NS = 16
NC = 2
NW = NS * NC
K = 64
NCHUNK = 160
E_PAD = NW * NCHUNK * K
EPT = NCHUNK * K
NBUF = 4
SUPER = 40
NSUPER = NCHUNK // SUPER
NGRP = SUPER // NBUF
NW_OUT = 10
ROWS_PT = N_NODES // NW_OUT


def _sc_body(ego, rowi, coli, vals, out,
             colb, rowb, valb, gbuf, acc, gsem, ssem):
    core = lax.axis_index("c")
    tid = lax.axis_index("s")
    wid = core * NS + tid

    def zrow(r, carry):
        for j in range(D_FEAT // 16):
            gbuf[r, pl.ds(j * 16, 16)] = jnp.zeros((16,), jnp.float32)
        return carry

    lax.fori_loop(0, NBUF * K, zrow, 0)

    @pl.when(tid < NW_OUT)
    def _zero():
        for off, nr in ((0, 256), (256, 256), (512, 256), (768, 232)):
            pltpu.sync_copy(gbuf.at[pl.ds(0, nr)],
                            acc.at[pl.ds(tid * ROWS_PT + off, nr)])

    plsc.subcore_barrier()

    def _drain_scatter(j):
        pltpu.make_async_copy(
            gbuf.at[pl.ds(j * K, K)], acc.at[rowb.at[0]], ssem.at[j]).wait()

    def superstep(s, carry):
        pltpu.sync_copy(coli.at[wid, pl.ds(s * SUPER, SUPER)], colb)
        pltpu.sync_copy(rowi.at[wid, pl.ds(s * SUPER, SUPER)], rowb)
        pltpu.sync_copy(vals.at[wid, pl.ds(s * SUPER, SUPER)], valb)

        def group(g, c1):
            @pl.when(jnp.logical_or(s > 0, g > 0))
            def _drain_prev():
                for j in range(NBUF):
                    _drain_scatter(j)

            gathers = []
            for j in range(NBUF):
                gathers.append(pltpu.async_copy(
                    ego.at[colb.at[g * NBUF + j]],
                    gbuf.at[pl.ds(j * K, K)], gsem.at[j]))
            for j in range(NBUF):
                gathers[j].wait()

                def scale(gg, c2, j=j):
                    v16 = valb[g * NBUF + j, pl.ds(gg * 16, 16)]
                    for i2 in range(16):
                        v = v16[i2]
                        e = j * K + gg * 16 + i2
                        for q in range(D_FEAT // 16):
                            gbuf[e, pl.ds(q * 16, 16)] = (
                                gbuf[e, pl.ds(q * 16, 16)] * v)
                    return c2

                lax.fori_loop(0, K // 16, scale, 0)
                pltpu.async_copy(
                    gbuf.at[pl.ds(j * K, K)],
                    acc.at[rowb.at[g * NBUF + j]], ssem.at[j], add=True)
            return c1

        lax.fori_loop(0, NGRP, group, 0)
        return carry

    lax.fori_loop(0, NSUPER, superstep, 0)
    for j in range(NBUF):
        _drain_scatter(j)

    plsc.subcore_barrier()

    @pl.when(tid < NW_OUT)
    def _writeout():
        pltpu.sync_copy(acc.at[pl.ds(tid * ROWS_PT, ROWS_PT)],
                        out.at[core, pl.ds(tid * ROWS_PT, ROWS_PT)])


def _combine_body(p_ref, o_ref):
    o_ref[...] = p_ref[0] + p_ref[1]


@jax.jit
def kernel(ego_embeddings, edge_index, adj_values):
    pad = E_PAD - N_EDGES
    spread = (jnp.arange(pad, dtype=jnp.int32) * 8) % N_NODES
    rowi = jnp.concatenate(
        [edge_index[0], spread]).reshape(NW, NCHUNK, K)
    coli = jnp.concatenate(
        [edge_index[1], spread]).reshape(NW, NCHUNK, K)
    vals = jnp.concatenate(
        [adj_values, jnp.zeros((pad,), jnp.float32)]).reshape(NW, NCHUNK, K)

    mesh = plsc.VectorSubcoreMesh(core_axis_name="c", subcore_axis_name="s")
    partials = pl.kernel(
        _sc_body,
        out_type=jax.ShapeDtypeStruct((NC, N_NODES, D_FEAT), jnp.float32),
        mesh=mesh,
        scratch_types=[
            pltpu.VMEM((SUPER, K), jnp.int32),
            pltpu.VMEM((SUPER, K), jnp.int32),
            pltpu.VMEM((SUPER, K), jnp.float32),
            pltpu.VMEM((NBUF * K, D_FEAT), jnp.float32),
            pltpu.VMEM_SHARED((N_NODES, D_FEAT), jnp.float32),
            pltpu.SemaphoreType.DMA((NBUF,)),
            pltpu.SemaphoreType.DMA((NBUF,)),
        ],
    )(ego_embeddings, rowi, coli, vals)

    rows_blk = 2000
    return pl.pallas_call(
        _combine_body,
        grid=(N_NODES // rows_blk,),
        in_specs=[pl.BlockSpec((NC, rows_blk, D_FEAT), lambda i: (0, i, 0))],
        out_specs=pl.BlockSpec((rows_blk, D_FEAT), lambda i: (i, 0)),
        out_shape=jax.ShapeDtypeStruct((N_NODES, D_FEAT), jnp.float32),
    )(partials)
